# Initial kernel scaffold; baseline (speedup 1.0000x reference)
import jax
import jax.numpy as jnp
from jax.experimental import pallas as pl


def kernel(x, edge_attr, W_atom, b_atom, W_rbf1, b_rbf1, W_rbf2, b_rbf2, Wq, bq, Wk, bk, Wv, bv, We, Wc, bc, Wmu, bmu, Wml, bml, g_ml, b_ml, g_ln, b_ln, g_bn, b_bn, Ws, bs, W_fc, b_fc, W_out, b_out, edge_index, batch):
    raise NotImplementedError("write your pallas kernel here")


# speedup vs baseline: 3.9518x; 3.9518x over previous
import functools

import jax
import jax.numpy as jnp
from jax import lax
from jax.experimental import pallas as pl
from jax.experimental.pallas import tpu as pltpu
from jax.experimental.pallas import tpu_sc as plsc

F32 = jnp.float32
NG = 32
EPS = 1e-5
BN_E = 1000
BN_N = 1000


def _softplus(t):
    return jnp.log1p(jnp.exp(-jnp.abs(t))) + jnp.maximum(t, 0.0)


def _silu(t):
    return t * lax.logistic(t)


def _prenode_body(x_ref, wa_ref, ba_ref, wq_ref, bq_ref, wk_ref, bk_ref,
                  wv_ref, bv_ref, h_ref, td_ref, ts_ref):
    h = jnp.dot(x_ref[...], wa_ref[...], preferred_element_type=F32) + ba_ref[...]
    q = jnp.dot(h, wq_ref[...], preferred_element_type=F32) + bq_ref[...]
    k = jnp.dot(h, wk_ref[...], preferred_element_type=F32) + bk_ref[...]
    v = jnp.dot(h, wv_ref[...], preferred_element_type=F32) + bv_ref[...]
    h_ref[...] = h
    td_ref[...] = jnp.concatenate([q, k, v, q * k], axis=-1)
    ts_ref[...] = jnp.concatenate([k, v], axis=-1)


def _ef_body(ea_ref, w1_ref, b1_ref, w2_ref, b2_ref, ef_ref):
    ea = ea_ref[...]
    bins = w1_ref.shape[0]
    dist = jnp.sqrt(jnp.sum(ea * ea, axis=1, keepdims=True))
    centers = lax.broadcasted_iota(jnp.int32, (1, bins), 1).astype(F32) * (
        8.0 / (bins - 1))
    gamma = (bins - 1) / 8.0
    diff = dist - centers
    rbf = jnp.exp(-gamma * diff * diff)
    t = _softplus(jnp.dot(rbf, w1_ref[...], preferred_element_type=F32) + b1_ref[...])
    ef_ref[...] = jnp.dot(t, w2_ref[...], preferred_element_type=F32) + b2_ref[...]


def _edge_body(gd_ref, gs_ref, ef_ref, we_ref, wmu_ref, bmu_ref, wml_ref,
               bml_ref, gln_ref, bln_ref, gml_ref, bml2_ref, msg_ref):
    gd = gd_ref[...]
    gs = gs_ref[...]
    c = we_ref.shape[0]
    q_i = gd[:, :c]
    v_i = gd[:, 2 * c:3 * c]
    qk_i = gd[:, 3 * c:]
    k_j = gs[:, :c]
    v_j = gs[:, c:]
    e = jnp.dot(ef_ref[...], we_ref[...], preferred_element_type=F32)
    inv_scale = 1.0 / (3.0 * c) ** 0.5
    a = jnp.concatenate([qk_i, q_i * k_j, q_i * e], axis=-1) * inv_scale
    m = jnp.mean(a, axis=-1, keepdims=True)
    var = jnp.mean(a * a, axis=-1, keepdims=True) - m * m
    gate = lax.logistic((a - m) * lax.rsqrt(var + EPS) * gln_ref[...] + bln_ref[...])
    msg0 = jnp.dot(jnp.concatenate([v_i, v_j, e], axis=-1), wmu_ref[...],
                   preferred_element_type=F32) + bmu_ref[...]
    msg = jnp.dot(msg0 * gate, wml_ref[...], preferred_element_type=F32) + bml_ref[...]
    m2 = jnp.mean(msg, axis=-1, keepdims=True)
    var2 = jnp.mean(msg * msg, axis=-1, keepdims=True) - m2 * m2
    out = (msg - m2) * lax.rsqrt(var2 + EPS) * gml_ref[...] + bml2_ref[...]
    msg_ref[...] = jnp.concatenate([out, jnp.zeros_like(out)], axis=-1)


def _node_update(a0_ref, a1_ref, h_ref, wc_ref, bc_ref, gbn_ref, bbn_ref,
                 ws_ref, bs_ref):
    c = wc_ref.shape[0]
    agg = a0_ref[...][:, :c] + a1_ref[...][:, :c]
    o = jnp.dot(agg, wc_ref[...], preferred_element_type=F32) + bc_ref[...]
    o = o * (1.0 / (1.0 + EPS) ** 0.5) * gbn_ref[...] + bbn_ref[...]
    o = _silu(o)
    return o + jnp.dot(h_ref[...], ws_ref[...], preferred_element_type=F32) + bs_ref[...]


def _node_body_tables(a0_ref, a1_ref, h_ref, wc_ref, bc_ref, gbn_ref, bbn_ref,
                      ws_ref, bs_ref, wq_ref, bq_ref, wk_ref, bk_ref, wv_ref,
                      bv_ref, hn_ref, td_ref, ts_ref):
    hn = _node_update(a0_ref, a1_ref, h_ref, wc_ref, bc_ref, gbn_ref, bbn_ref,
                      ws_ref, bs_ref)
    q = jnp.dot(hn, wq_ref[...], preferred_element_type=F32) + bq_ref[...]
    k = jnp.dot(hn, wk_ref[...], preferred_element_type=F32) + bk_ref[...]
    v = jnp.dot(hn, wv_ref[...], preferred_element_type=F32) + bv_ref[...]
    hn_ref[...] = hn
    td_ref[...] = jnp.concatenate([q, k, v, q * k], axis=-1)
    ts_ref[...] = jnp.concatenate([k, v], axis=-1)


def _node_body_last(a0_ref, a1_ref, h_ref, wc_ref, bc_ref, gbn_ref, bbn_ref,
                    ws_ref, bs_ref, hn_ref):
    hn_ref[...] = _node_update(a0_ref, a1_ref, h_ref, wc_ref, bc_ref, gbn_ref,
                               bbn_ref, ws_ref, bs_ref)


def _pool_body(h_ref, b_ref, wfc_ref, bfc_ref, wout_ref, bout_ref, out_ref,
               acc_ref):
    pid = pl.program_id(0)

    @pl.when(pid == 0)
    def _():
        acc_ref[...] = jnp.zeros_like(acc_ref)

    bb = b_ref[...]
    io = lax.broadcasted_iota(jnp.int32, (bb.shape[0], NG), 1)
    oh = (bb == io).astype(F32)
    acc_ref[...] += lax.dot_general(oh, h_ref[...], (((0,), (0,)), ((), ())),
                                    preferred_element_type=F32)

    @pl.when(pid == pl.num_programs(0) - 1)
    def _():
        t = _silu(jnp.dot(acc_ref[...], wfc_ref[...],
                          preferred_element_type=F32) + bfc_ref[...])
        out_ref[...] = jnp.dot(t, wout_ref[...],
                               preferred_element_type=F32) + bout_ref[...]


def _row_spec(b, d):
    return pl.BlockSpec((b, d), lambda i: (i, 0))


def _full_spec(shape):
    return pl.BlockSpec(shape, lambda i: tuple(0 for _ in shape))


def _tc_prenode(x, wa, ba, wq, bq, wk, bk, wv, bv):
    n, aif = x.shape
    c = wa.shape[1]
    grid = (n // BN_N,)
    return pl.pallas_call(
        _prenode_body,
        grid=grid,
        in_specs=[_row_spec(BN_N, aif)] + [
            _full_spec(w.shape) for w in (wa, ba, wq, bq, wk, bk, wv, bv)],
        out_specs=[_row_spec(BN_N, c), _row_spec(BN_N, 4 * c),
                   _row_spec(BN_N, 2 * c)],
        out_shape=[jax.ShapeDtypeStruct((n, c), F32),
                   jax.ShapeDtypeStruct((n, 4 * c), F32),
                   jax.ShapeDtypeStruct((n, 2 * c), F32)],
    )(x, wa, ba, wq, bq, wk, bk, wv, bv)


def _tc_ef(ea, w1, b1, w2, b2):
    e = ea.shape[0]
    c = w1.shape[1]
    return pl.pallas_call(
        _ef_body,
        grid=(e // BN_E,),
        in_specs=[_row_spec(BN_E, ea.shape[1])] + [
            _full_spec(w.shape) for w in (w1, b1, w2, b2)],
        out_specs=_row_spec(BN_E, c),
        out_shape=jax.ShapeDtypeStruct((e, c), F32),
    )(ea, w1, b1, w2, b2)


def _tc_edge(gd, gs, ef, we, wmu, bmu, wml, bml, gln, bln, gml, bml2):
    e = gd.shape[0]
    c = we.shape[0]
    return pl.pallas_call(
        _edge_body,
        grid=(e // BN_E,),
        in_specs=[_row_spec(BN_E, 4 * c), _row_spec(BN_E, 2 * c),
                  _row_spec(BN_E, c)] + [
            _full_spec(w.shape)
            for w in (we, wmu, bmu, wml, bml, gln, bln, gml, bml2)],
        out_specs=_row_spec(BN_E, 2 * c),
        out_shape=jax.ShapeDtypeStruct((e, 2 * c), F32),
    )(gd, gs, ef, we, wmu, bmu, wml, bml, gln, bln, gml, bml2)


def _tc_node(parts, h, wc, bc, gbn, bbn, ws, bs, nxt):
    n, c = h.shape
    a0 = parts[0]
    a1 = parts[1]
    base_in = [_row_spec(BN_N, a0.shape[1])] * 2 + [_row_spec(BN_N, c)] + [
        _full_spec(w.shape) for w in (wc, bc, gbn, bbn, ws, bs)]
    if nxt is None:
        return pl.pallas_call(
            _node_body_last,
            grid=(n // BN_N,),
            in_specs=base_in,
            out_specs=_row_spec(BN_N, c),
            out_shape=jax.ShapeDtypeStruct((n, c), F32),
        )(a0, a1, h, wc, bc, gbn, bbn, ws, bs)
    wq, bq, wk, bk, wv, bv = nxt
    return pl.pallas_call(
        _node_body_tables,
        grid=(n // BN_N,),
        in_specs=base_in + [
            _full_spec(w.shape) for w in (wq, bq, wk, bk, wv, bv)],
        out_specs=[_row_spec(BN_N, c), _row_spec(BN_N, 4 * c),
                   _row_spec(BN_N, 2 * c)],
        out_shape=[jax.ShapeDtypeStruct((n, c), F32),
                   jax.ShapeDtypeStruct((n, 4 * c), F32),
                   jax.ShapeDtypeStruct((n, 2 * c), F32)],
    )(a0, a1, h, wc, bc, gbn, bbn, ws, bs, wq, bq, wk, bk, wv, bv)


def _tc_pool(h, batch2, wfc, bfc, wout, bout):
    n, c = h.shape
    fc = wfc.shape[1]
    return pl.pallas_call(
        _pool_body,
        grid=(n // BN_N,),
        in_specs=[_row_spec(BN_N, c), _row_spec(BN_N, 1)] + [
            _full_spec(w.shape) for w in (wfc, bfc, wout, bout)],
        out_specs=_full_spec((NG, 1)),
        out_shape=jax.ShapeDtypeStruct((NG, 1), F32),
        scratch_shapes=[pltpu.VMEM((NG, c), F32)],
    )(h, batch2, wfc, bfc, wout, bout)


@functools.lru_cache(maxsize=None)
def _make_sc_gather(n, e, d1, d2):
    info = plsc.get_sparse_core_info()
    nc, ns = info.num_cores, info.num_subcores
    nw = nc * ns
    epw = e // nw
    ch = 200
    nchunk = epw // ch
    mesh = plsc.VectorSubcoreMesh(core_axis_name="c", subcore_axis_name="s")

    @functools.partial(
        pl.kernel,
        out_type=[jax.ShapeDtypeStruct((e, d1), F32),
                  jax.ShapeDtypeStruct((e, d2), F32)],
        mesh=mesh,
        scratch_types=[
            pltpu.VMEM((ch,), jnp.int32),
            pltpu.VMEM((ch,), jnp.int32),
            pltpu.VMEM((ch, d1), F32),
            pltpu.VMEM((ch, d2), F32),
            pltpu.SemaphoreType.DMA,
            pltpu.SemaphoreType.DMA,
        ],
    )
    def gk(td_hbm, ts_hbm, dst_hbm, src_hbm, gd_hbm, gs_hbm,
           idxd, idxs, rowd, rows, semd, sems):
        wid = lax.axis_index("s") * nc + lax.axis_index("c")
        base0 = wid * epw

        def step(j, carry):
            base = pl.multiple_of(base0 + j * ch, 8)
            pltpu.sync_copy(dst_hbm.at[pl.ds(base, ch)], idxd)
            pltpu.sync_copy(src_hbm.at[pl.ds(base, ch)], idxs)
            cp1 = pltpu.async_copy(td_hbm.at[idxd], rowd, semd)
            cp2 = pltpu.async_copy(ts_hbm.at[idxs], rows, sems)
            cp1.wait()
            cp2.wait()
            pltpu.sync_copy(rowd, gd_hbm.at[pl.ds(base, ch)])
            pltpu.sync_copy(rows, gs_hbm.at[pl.ds(base, ch)])
            return carry

        lax.fori_loop(0, nchunk, step, 0)

    return gk


@functools.lru_cache(maxsize=None)
def _make_sc_scatter(n, e, d):
    info = plsc.get_sparse_core_info()
    nc, ns = info.num_cores, info.num_subcores
    nw = nc * ns
    epw = e // nw
    ch = 200
    nchunk = epw // ch
    zch = 80
    nzch = n // zch
    mesh = plsc.VectorSubcoreMesh(core_axis_name="c", subcore_axis_name="s")

    @functools.partial(
        pl.kernel,
        out_type=jax.ShapeDtypeStruct((nc, n, d), F32),
        mesh=mesh,
        scratch_types=[
            pltpu.VMEM((ch,), jnp.int32),
            pltpu.VMEM((ch, d), F32),
            pltpu.VMEM((zch, d), F32),
            pltpu.VMEM_SHARED((n, d), F32),
            pltpu.SemaphoreType.DMA,
        ],
    )
    def sk(msg_hbm, dst_hbm, out_hbm, idxb, rows, zbuf, acc, sem):
        c = lax.axis_index("c")
        s = lax.axis_index("s")
        wid = s * nc + c
        base0 = wid * epw

        z16 = jnp.zeros((16,), F32)

        def zstep(i, carry):
            for t in range(d // 16):
                zbuf[i, pl.ds(t * 16, 16)] = z16
            return carry

        lax.fori_loop(0, zch, zstep, 0)

        def zinit(j, carry):
            ck = s + j * ns

            @pl.when(ck < nzch)
            def _():
                rbase = pl.multiple_of(ck * zch, 8)
                pltpu.sync_copy(zbuf, acc.at[pl.ds(rbase, zch)])

            return carry

        lax.fori_loop(0, (nzch + ns - 1) // ns, zinit, 0)
        plsc.subcore_barrier()

        def step(j, carry):
            base = pl.multiple_of(base0 + j * ch, 8)
            pltpu.sync_copy(msg_hbm.at[pl.ds(base, ch)], rows)
            pltpu.sync_copy(dst_hbm.at[pl.ds(base, ch)], idxb)
            pltpu.sync_copy(rows, acc.at[idxb], add=True)
            return carry

        lax.fori_loop(0, nchunk, step, 0)
        plsc.subcore_barrier()

        def flush(j, carry):
            ck = s + j * ns

            @pl.when(ck < nzch)
            def _():
                rbase = pl.multiple_of(ck * zch, 8)
                pltpu.sync_copy(acc.at[pl.ds(rbase, zch)], zbuf)
                pltpu.sync_copy(zbuf, out_hbm.at[c, pl.ds(rbase, zch)])

            return carry

        lax.fori_loop(0, (nzch + ns - 1) // ns, flush, 0)

    return sk


def _sc_gather(td, ts, dst, src):
    n, d1 = td.shape
    d2 = ts.shape[1]
    e = dst.shape[0]
    return _make_sc_gather(n, e, d1, d2)(td, ts, dst, src)


def _sc_scatter(msg, dst, n):
    e, d = msg.shape
    return _make_sc_scatter(n, e, d)(msg, dst)


def kernel(x, edge_attr, W_atom, b_atom, W_rbf1, b_rbf1, W_rbf2, b_rbf2,
           Wq, bq, Wk, bk, Wv, bv, We, Wc, bc, Wmu, bmu, Wml, bml,
           g_ml, b_ml, g_ln, b_ln, g_bn, b_bn, Ws, bs, W_fc, b_fc,
           W_out, b_out, edge_index, batch):
    n = x.shape[0]
    num_layers = Wq.shape[0]

    r = lambda w: w.reshape(1, -1)

    src = edge_index[0]
    dst = edge_index[1]

    h, td, ts = _tc_prenode(x, W_atom, r(b_atom), Wq[0], r(bq[0]), Wk[0],
                            r(bk[0]), Wv[0], r(bv[0]))
    ef = _tc_ef(edge_attr, W_rbf1, r(b_rbf1), W_rbf2, r(b_rbf2))

    for i in range(num_layers):
        gd, gs = _sc_gather(td, ts, dst, src)
        msg = _tc_edge(gd, gs, ef, We[i], Wmu[i], r(bmu[i]), Wml[i],
                       r(bml[i]), r(g_ln[i]), r(b_ln[i]), r(g_ml[i]),
                       r(b_ml[i]))
        parts = _sc_scatter(msg, dst, n)
        if i + 1 < num_layers:
            nxt = (Wq[i + 1], r(bq[i + 1]), Wk[i + 1], r(bk[i + 1]),
                   Wv[i + 1], r(bv[i + 1]))
            h, td, ts = _tc_node(parts, h, Wc[i], r(bc[i]), r(g_bn[i]),
                                 r(b_bn[i]), Ws[i], r(bs[i]), nxt)
        else:
            h = _tc_node(parts, h, Wc[i], r(bc[i]), r(g_bn[i]), r(b_bn[i]),
                         Ws[i], r(bs[i]), None)

    out = _tc_pool(h, batch.reshape(-1, 1), W_fc, r(b_fc), W_out, r(b_out))
    return jnp.squeeze(out)
